# baseline (speedup 1.0000x reference)
import functools

import jax
import jax.numpy as jnp
from jax import lax
from jax.experimental import pallas as pl
from jax.experimental.pallas import tpu as pltpu
from jax.experimental.pallas import tpu_sc as plsc

N = 10000
E = 320000
DIM = 128

NC = 2
NS = 16
NW = NC * NS
LANES = 16

CHUNK = 128
CH0 = 116
CH1 = 42
CHMAX = max(CH0, CH1)
VSLOTS = 4
E_PAD = NS * CHUNK * (CH0 + CH1)

N_PAD = 10112
ROWS_PER_SUB = N_PAD // NS
ZERO_BLOCKS = ROWS_PER_SUB // CHUNK
ZERO_TAIL = ROWS_PER_SUB - ZERO_BLOCKS * CHUNK

ROW_BLK = 1000


def _mm_body(x_ref, w_ref, o_ref):
    o_ref[...] = jnp.dot(x_ref[...], w_ref[...],
                         preferred_element_type=jnp.float32)


def _support_matmul(x, w):
    return pl.pallas_call(
        _mm_body,
        grid=(N // ROW_BLK,),
        in_specs=[
            pl.BlockSpec((ROW_BLK, DIM), lambda i: (i, 0)),
            pl.BlockSpec((DIM, DIM), lambda i: (0, 0)),
        ],
        out_specs=pl.BlockSpec((ROW_BLK, DIM), lambda i: (i, 0)),
        out_shape=jax.ShapeDtypeStruct((N, DIM), jnp.float32),
    )(x, w)


NSTREAM = 4
QROWS = CHUNK // NSTREAM


def _sc_body(support_hbm, src_hbm, dst_hbm, val_hbm, out_hbm,
             src_v, dst_v, val_r, rows_v, acc_sh,
             sem, sem1, sem2, sem3, vsem):
    c = lax.axis_index("c")
    s = lax.axis_index("s")
    wid = c * NS + s
    nchunks = jnp.where(c == 0, CH0, CH1)

    def copy_val(j, q):
        pltpu.async_copy(val_hbm.at[wid, j], val_r.at[q], vsem.at[q])

    def wait_val(q):
        pltpu.make_async_copy(val_hbm.at[wid, 0], val_r.at[q],
                              vsem.at[q]).wait()

    pltpu.sync_copy(src_hbm.at[wid], src_v)
    pltpu.sync_copy(dst_hbm.at[wid], dst_v)
    for q in range(VSLOTS - 1):
        copy_val(q, q)

    zeros = jnp.zeros((LANES,), jnp.float32)

    def zero_row(r, carry):
        for k in range(DIM // LANES):
            rows_v[r, pl.ds(k * LANES, LANES)] = zeros
        return carry

    lax.fori_loop(0, CHUNK, zero_row, 0)
    for b in range(ZERO_BLOCKS):
        pltpu.sync_copy(
            rows_v, acc_sh.at[pl.ds(s * ROWS_PER_SUB + b * CHUNK, CHUNK)])
    pltpu.sync_copy(
        rows_v.at[pl.ds(0, ZERO_TAIL)],
        acc_sh.at[pl.ds(s * ROWS_PER_SUB + ZERO_BLOCKS * CHUNK, ZERO_TAIL)])
    plsc.subcore_barrier()

    def chunk_step(j, carry):
        sems = (sem, sem1, sem2, sem3)
        descs = []
        for q in range(NSTREAM):
            descs.append(pltpu.async_copy(
                support_hbm.at[src_v.at[j, pl.ds(q * QROWS, QROWS)]],
                rows_v.at[pl.ds(q * QROWS, QROWS)], sems[q]))
        for d in descs:
            d.wait()

        vq = lax.rem(j, VSLOTS)
        wait_val(vq)

        def scale_group(g, c2):
            vv = val_r[vq, pl.ds(g * LANES, LANES)]
            base = g * LANES
            for e16 in range(LANES):
                v = vv[e16]
                for k in range(DIM // LANES):
                    sl = pl.ds(k * LANES, LANES)
                    rows_v[base + e16, sl] = rows_v[base + e16, sl] * v
            return c2

        lax.fori_loop(0, CHUNK // LANES, scale_group, 0)

        @pl.when(j + VSLOTS - 1 < nchunks)
        def _():
            copy_val(j + VSLOTS - 1, lax.rem(j + VSLOTS - 1, VSLOTS))

        pltpu.sync_copy(rows_v, acc_sh.at[dst_v.at[j]], add=True)
        return carry

    lax.fori_loop(0, nchunks, chunk_step, 0)
    plsc.subcore_barrier()

    pltpu.sync_copy(acc_sh.at[pl.ds(s * ROWS_PER_SUB, ROWS_PER_SUB)],
                    out_hbm.at[c, pl.ds(s * ROWS_PER_SUB, ROWS_PER_SUB)])


_sc_scatter = functools.partial(
    pl.kernel,
    out_type=jax.ShapeDtypeStruct((NC, N_PAD, DIM), jnp.float32),
    mesh=plsc.VectorSubcoreMesh(core_axis_name="c", subcore_axis_name="s",
                                num_cores=NC, num_subcores=NS),
    scratch_types=[
        pltpu.VMEM((CHMAX, CHUNK), jnp.int32),
        pltpu.VMEM((CHMAX, CHUNK), jnp.int32),
        pltpu.VMEM((VSLOTS, CHUNK), jnp.float32),
        pltpu.VMEM((CHUNK, DIM), jnp.float32),
        pltpu.VMEM_SHARED((N_PAD, DIM), jnp.float32),
        pltpu.SemaphoreType.DMA,
        pltpu.SemaphoreType.DMA,
        pltpu.SemaphoreType.DMA,
        pltpu.SemaphoreType.DMA,
        pltpu.SemaphoreType.DMA((VSLOTS,)),
    ],
)(_sc_body)


def _epilogue_body(a0_ref, a1_ref, bg_ref, w2_ref, b2_ref, o_ref):
    x1 = jnp.maximum(a0_ref[...] + a1_ref[...] + bg_ref[...], 0.0)
    x2 = jnp.dot(x1, w2_ref[...], preferred_element_type=jnp.float32)
    x2 = x2 + b2_ref[...]
    nrm = jnp.sqrt(jnp.sum(x2 * x2, axis=1, keepdims=True))
    o_ref[...] = x2 / nrm


def _epilogue(a0, a1, b_gc, w2, b2):
    return pl.pallas_call(
        _epilogue_body,
        grid=(N // ROW_BLK,),
        in_specs=[
            pl.BlockSpec((ROW_BLK, DIM), lambda i: (i, 0)),
            pl.BlockSpec((ROW_BLK, DIM), lambda i: (i, 0)),
            pl.BlockSpec((1, DIM), lambda i: (0, 0)),
            pl.BlockSpec((DIM, DIM), lambda i: (0, 0)),
            pl.BlockSpec((1, DIM), lambda i: (0, 0)),
        ],
        out_specs=pl.BlockSpec((ROW_BLK, DIM), lambda i: (i, 0)),
        out_shape=jax.ShapeDtypeStruct((N, DIM), jnp.float32),
    )(a0, a1, b_gc, w2, b2)


def _shard_edges(arr):
    flat = jnp.pad(arr, (0, E_PAD - E))
    n0 = NS * CH0 * CHUNK
    p0 = flat[:n0].reshape(NS, CH0, CHUNK)
    p0 = jnp.pad(p0, ((0, 0), (0, CHMAX - CH0), (0, 0)))
    p1 = flat[n0:].reshape(NS, CH1, CHUNK)
    p1 = jnp.pad(p1, ((0, 0), (0, CHMAX - CH1), (0, 0)))
    return jnp.concatenate([p0, p1], axis=0)


def kernel(x, edge_index, edge_vals, W_gc, b_gc, W2, b2):
    support = _support_matmul(x, W_gc)

    src = _shard_edges(edge_index[0])
    dst = _shard_edges(edge_index[1])
    val = _shard_edges(edge_vals)

    partials = _sc_scatter(support, src, dst, val)

    out = _epilogue(partials[0, :N], partials[1, :N],
                    b_gc.reshape(1, DIM), W2.reshape(DIM, DIM),
                    b2.reshape(1, DIM))
    return out


# speedup vs baseline: 1.0129x; 1.0129x over previous
import functools

import jax
import jax.numpy as jnp
from jax import lax
from jax.experimental import pallas as pl
from jax.experimental.pallas import tpu as pltpu
from jax.experimental.pallas import tpu_sc as plsc

N = 10000
E = 320000
DIM = 128

NC = 2
NS = 16
NW = NC * NS
LANES = 16

CHUNK = 128
CH0 = 108
CH1 = 50
CHMAX = max(CH0, CH1)
VSLOTS = 4
E_PAD = NS * CHUNK * (CH0 + CH1)

N_PAD = 10112
ROWS_PER_SUB = N_PAD // NS
ZERO_BLOCKS = ROWS_PER_SUB // CHUNK
ZERO_TAIL = ROWS_PER_SUB - ZERO_BLOCKS * CHUNK

ROW_BLK = 1000


def _mm_body(x_ref, w_ref, o_ref):
    o_ref[...] = jnp.dot(x_ref[...], w_ref[...],
                         preferred_element_type=jnp.float32)


def _support_matmul(x, w):
    return pl.pallas_call(
        _mm_body,
        grid=(N // ROW_BLK,),
        in_specs=[
            pl.BlockSpec((ROW_BLK, DIM), lambda i: (i, 0)),
            pl.BlockSpec((DIM, DIM), lambda i: (0, 0)),
        ],
        out_specs=pl.BlockSpec((ROW_BLK, DIM), lambda i: (i, 0)),
        out_shape=jax.ShapeDtypeStruct((N, DIM), jnp.float32),
    )(x, w)


NSTREAM = 4
QROWS = CHUNK // NSTREAM


def _sc_body(support_hbm, src_hbm, dst_hbm, val_hbm, out_hbm,
             src_v, dst_v, val_r, rows_v, acc_sh,
             sem, sem1, sem2, sem3, vsem):
    c = lax.axis_index("c")
    s = lax.axis_index("s")
    wid = c * NS + s
    nchunks = jnp.where(c == 0, CH0, CH1)

    def copy_val(j, q):
        pltpu.async_copy(val_hbm.at[wid, j], val_r.at[q], vsem.at[q])

    def wait_val(q):
        pltpu.make_async_copy(val_hbm.at[wid, 0], val_r.at[q],
                              vsem.at[q]).wait()

    pltpu.sync_copy(src_hbm.at[wid], src_v)
    pltpu.sync_copy(dst_hbm.at[wid], dst_v)
    for q in range(VSLOTS - 1):
        copy_val(q, q)

    zeros = jnp.zeros((LANES,), jnp.float32)

    def zero_row(r, carry):
        for k in range(DIM // LANES):
            rows_v[r, pl.ds(k * LANES, LANES)] = zeros
        return carry

    lax.fori_loop(0, CHUNK, zero_row, 0)
    for b in range(ZERO_BLOCKS):
        pltpu.sync_copy(
            rows_v, acc_sh.at[pl.ds(s * ROWS_PER_SUB + b * CHUNK, CHUNK)])
    pltpu.sync_copy(
        rows_v.at[pl.ds(0, ZERO_TAIL)],
        acc_sh.at[pl.ds(s * ROWS_PER_SUB + ZERO_BLOCKS * CHUNK, ZERO_TAIL)])
    plsc.subcore_barrier()

    def chunk_step(j, carry):
        sems = (sem, sem1, sem2, sem3)
        descs = []
        for q in range(NSTREAM):
            descs.append(pltpu.async_copy(
                support_hbm.at[src_v.at[j, pl.ds(q * QROWS, QROWS)]],
                rows_v.at[pl.ds(q * QROWS, QROWS)], sems[q]))
        for d in descs:
            d.wait()

        vq = lax.rem(j, VSLOTS)
        wait_val(vq)

        def scale_group(g, c2):
            vv = val_r[vq, pl.ds(g * LANES, LANES)]
            base = g * LANES
            for e16 in range(LANES):
                v = vv[e16]
                for k in range(DIM // LANES):
                    sl = pl.ds(k * LANES, LANES)
                    rows_v[base + e16, sl] = rows_v[base + e16, sl] * v
            return c2

        lax.fori_loop(0, CHUNK // LANES, scale_group, 0)

        @pl.when(j + VSLOTS - 1 < nchunks)
        def _():
            copy_val(j + VSLOTS - 1, lax.rem(j + VSLOTS - 1, VSLOTS))

        pltpu.sync_copy(rows_v, acc_sh.at[dst_v.at[j]], add=True)
        return carry

    lax.fori_loop(0, nchunks, chunk_step, 0)
    plsc.subcore_barrier()

    pltpu.sync_copy(acc_sh.at[pl.ds(s * ROWS_PER_SUB, ROWS_PER_SUB)],
                    out_hbm.at[c, pl.ds(s * ROWS_PER_SUB, ROWS_PER_SUB)])


_sc_scatter = functools.partial(
    pl.kernel,
    out_type=jax.ShapeDtypeStruct((NC, N_PAD, DIM), jnp.float32),
    mesh=plsc.VectorSubcoreMesh(core_axis_name="c", subcore_axis_name="s",
                                num_cores=NC, num_subcores=NS),
    scratch_types=[
        pltpu.VMEM((CHMAX, CHUNK), jnp.int32),
        pltpu.VMEM((CHMAX, CHUNK), jnp.int32),
        pltpu.VMEM((VSLOTS, CHUNK), jnp.float32),
        pltpu.VMEM((CHUNK, DIM), jnp.float32),
        pltpu.VMEM_SHARED((N_PAD, DIM), jnp.float32),
        pltpu.SemaphoreType.DMA,
        pltpu.SemaphoreType.DMA,
        pltpu.SemaphoreType.DMA,
        pltpu.SemaphoreType.DMA,
        pltpu.SemaphoreType.DMA((VSLOTS,)),
    ],
)(_sc_body)


def _epilogue_body(a0_ref, a1_ref, bg_ref, w2_ref, b2_ref, o_ref):
    x1 = jnp.maximum(a0_ref[...] + a1_ref[...] + bg_ref[...], 0.0)
    x2 = jnp.dot(x1, w2_ref[...], preferred_element_type=jnp.float32)
    x2 = x2 + b2_ref[...]
    nrm = jnp.sqrt(jnp.sum(x2 * x2, axis=1, keepdims=True))
    o_ref[...] = x2 / nrm


def _epilogue(a0, a1, b_gc, w2, b2):
    return pl.pallas_call(
        _epilogue_body,
        grid=(N // ROW_BLK,),
        in_specs=[
            pl.BlockSpec((ROW_BLK, DIM), lambda i: (i, 0)),
            pl.BlockSpec((ROW_BLK, DIM), lambda i: (i, 0)),
            pl.BlockSpec((1, DIM), lambda i: (0, 0)),
            pl.BlockSpec((DIM, DIM), lambda i: (0, 0)),
            pl.BlockSpec((1, DIM), lambda i: (0, 0)),
        ],
        out_specs=pl.BlockSpec((ROW_BLK, DIM), lambda i: (i, 0)),
        out_shape=jax.ShapeDtypeStruct((N, DIM), jnp.float32),
    )(a0, a1, b_gc, w2, b2)


def _shard_edges(arr):
    flat = jnp.pad(arr, (0, E_PAD - E))
    n0 = NS * CH0 * CHUNK
    p0 = flat[:n0].reshape(NS, CH0, CHUNK)
    p0 = jnp.pad(p0, ((0, 0), (0, CHMAX - CH0), (0, 0)))
    p1 = flat[n0:].reshape(NS, CH1, CHUNK)
    p1 = jnp.pad(p1, ((0, 0), (0, CHMAX - CH1), (0, 0)))
    return jnp.concatenate([p0, p1], axis=0)


def kernel(x, edge_index, edge_vals, W_gc, b_gc, W2, b2):
    support = _support_matmul(x, W_gc)

    src = _shard_edges(edge_index[0])
    dst = _shard_edges(edge_index[1])
    val = _shard_edges(edge_vals)

    partials = _sc_scatter(support, src, dst, val)

    out = _epilogue(partials[0, :N], partials[1, :N],
                    b_gc.reshape(1, DIM), W2.reshape(DIM, DIM),
                    b2.reshape(1, DIM))
    return out
